# baseline (speedup 1.0000x reference)
import functools

import jax
import jax.numpy as jnp
from jax import lax
from jax.experimental import pallas as pl
from jax.experimental.pallas import tpu as pltpu
from jax.experimental.pallas import tpu_sc as plsc

_NUM_WORKERS = 32
_CHUNK = 16


def _emb_kernel_body(n_chunks, rows_per_worker, seq, x_hbm, table_hbm, out_hbm,
                     idx_v, rows0, rows1, sem0, sem1):
    wid = lax.axis_index("s") * 2 + lax.axis_index("c")
    base = wid * rows_per_worker
    w_per_row = seq // rows_per_worker
    pltpu.sync_copy(
        x_hbm.at[wid // w_per_row,
                 pl.ds((wid % w_per_row) * rows_per_worker, rows_per_worker)],
        idx_v)

    bufs = (rows0, rows1)
    sems = (sem0, sem1)
    handles = [None, None]
    handles[0] = pltpu.async_copy(
        table_hbm.at[idx_v.at[pl.ds(0, _CHUNK)]], bufs[0], sems[0])
    for c in range(n_chunks):
        nxt = c + 1
        if nxt < n_chunks:
            handles[nxt % 2] = pltpu.async_copy(
                table_hbm.at[idx_v.at[pl.ds(nxt * _CHUNK, _CHUNK)]],
                bufs[nxt % 2], sems[nxt % 2])
        handles[c % 2].wait()
        pltpu.sync_copy(bufs[c % 2], out_hbm.at[pl.ds(base + c * _CHUNK, _CHUNK)])


def kernel(x, table):
    batch, seq = x.shape
    _, d_model = table.shape
    n = batch * seq
    rows_per_worker = n // _NUM_WORKERS
    n_chunks = rows_per_worker // _CHUNK

    mesh = plsc.VectorSubcoreMesh(core_axis_name="c", subcore_axis_name="s")
    emb = pl.kernel(
        functools.partial(_emb_kernel_body, n_chunks, rows_per_worker, seq),
        mesh=mesh,
        out_type=jax.ShapeDtypeStruct((n, d_model), jnp.float32),
        scratch_types=[
            pltpu.VMEM((rows_per_worker,), jnp.int32),
            pltpu.VMEM((_CHUNK, d_model), jnp.float32),
            pltpu.VMEM((_CHUNK, d_model), jnp.float32),
            pltpu.SemaphoreType.DMA,
            pltpu.SemaphoreType.DMA,
        ],
    )
    out = emb(x.astype(jnp.int32), table)
    return out.reshape(batch, seq, d_model)


# speedup vs baseline: 1.0157x; 1.0157x over previous
import functools

import jax
import jax.numpy as jnp
from jax import lax
from jax.experimental import pallas as pl
from jax.experimental.pallas import tpu as pltpu
from jax.experimental.pallas import tpu_sc as plsc

_NUM_WORKERS = 32
_CHUNK = 32


def _emb_kernel_body(n_chunks, rows_per_worker, seq, x_hbm, table_hbm, out_hbm,
                     idx_v, rows0, rows1, sem0, sem1):
    wid = lax.axis_index("s") * 2 + lax.axis_index("c")
    base = wid * rows_per_worker
    w_per_row = seq // rows_per_worker
    pltpu.sync_copy(
        x_hbm.at[wid // w_per_row,
                 pl.ds((wid % w_per_row) * rows_per_worker, rows_per_worker)],
        idx_v)

    bufs = (rows0, rows1)
    sems = (sem0, sem1)
    handles = [None, None]
    handles[0] = pltpu.async_copy(
        table_hbm.at[idx_v.at[pl.ds(0, _CHUNK)]], bufs[0], sems[0])
    for c in range(n_chunks):
        nxt = c + 1
        if nxt < n_chunks:
            handles[nxt % 2] = pltpu.async_copy(
                table_hbm.at[idx_v.at[pl.ds(nxt * _CHUNK, _CHUNK)]],
                bufs[nxt % 2], sems[nxt % 2])
        handles[c % 2].wait()
        pltpu.sync_copy(bufs[c % 2], out_hbm.at[pl.ds(base + c * _CHUNK, _CHUNK)])


def kernel(x, table):
    batch, seq = x.shape
    _, d_model = table.shape
    n = batch * seq
    rows_per_worker = n // _NUM_WORKERS
    n_chunks = rows_per_worker // _CHUNK

    mesh = plsc.VectorSubcoreMesh(core_axis_name="c", subcore_axis_name="s")
    emb = pl.kernel(
        functools.partial(_emb_kernel_body, n_chunks, rows_per_worker, seq),
        mesh=mesh,
        out_type=jax.ShapeDtypeStruct((n, d_model), jnp.float32),
        scratch_types=[
            pltpu.VMEM((rows_per_worker,), jnp.int32),
            pltpu.VMEM((_CHUNK, d_model), jnp.float32),
            pltpu.VMEM((_CHUNK, d_model), jnp.float32),
            pltpu.SemaphoreType.DMA,
            pltpu.SemaphoreType.DMA,
        ],
    )
    out = emb(x.astype(jnp.int32), table)
    return out.reshape(batch, seq, d_model)


# speedup vs baseline: 1.0254x; 1.0095x over previous
import functools

import jax
import jax.numpy as jnp
from jax import lax
from jax.experimental import pallas as pl
from jax.experimental.pallas import tpu as pltpu
from jax.experimental.pallas import tpu_sc as plsc

_NUM_WORKERS = 32
_CHUNK = 32


def _emb_kernel_body(n_chunks, rows_per_worker, seq, x_hbm, table_hbm, out_hbm,
                     idx_v, rows0, rows1, sem0, sem1):
    wid = lax.axis_index("s") * 2 + lax.axis_index("c")
    base = wid * rows_per_worker
    w_per_row = seq // rows_per_worker
    pltpu.sync_copy(
        x_hbm.at[wid // w_per_row,
                 pl.ds((wid % w_per_row) * rows_per_worker, rows_per_worker)],
        idx_v)

    def gather(c, buf, sem):
        off = pl.multiple_of(c * _CHUNK, _CHUNK)
        pltpu.async_copy(table_hbm.at[idx_v.at[pl.ds(off, _CHUNK)]], buf, sem)

    def wait_out(c, buf, sem):
        off = pl.multiple_of(c * _CHUNK, _CHUNK)
        pltpu.make_async_copy(
            table_hbm.at[idx_v.at[pl.ds(off, _CHUNK)]], buf, sem).wait()
        pltpu.sync_copy(buf, out_hbm.at[pl.ds(base + off, _CHUNK)])

    gather(0, rows0, sem0)
    gather(1, rows1, sem1)

    @pl.loop(0, n_chunks // 2 - 1)
    def _(g):
        c0 = 2 * g
        wait_out(c0, rows0, sem0)
        gather(c0 + 2, rows0, sem0)
        wait_out(c0 + 1, rows1, sem1)
        gather(c0 + 3, rows1, sem1)

    wait_out(n_chunks - 2, rows0, sem0)
    wait_out(n_chunks - 1, rows1, sem1)


def kernel(x, table):
    batch, seq = x.shape
    _, d_model = table.shape
    n = batch * seq
    rows_per_worker = n // _NUM_WORKERS
    n_chunks = rows_per_worker // _CHUNK

    mesh = plsc.VectorSubcoreMesh(core_axis_name="c", subcore_axis_name="s")
    emb = pl.kernel(
        functools.partial(_emb_kernel_body, n_chunks, rows_per_worker, seq),
        mesh=mesh,
        out_type=jax.ShapeDtypeStruct((n, d_model), jnp.float32),
        scratch_types=[
            pltpu.VMEM((rows_per_worker,), jnp.int32),
            pltpu.VMEM((_CHUNK, d_model), jnp.float32),
            pltpu.VMEM((_CHUNK, d_model), jnp.float32),
            pltpu.SemaphoreType.DMA,
            pltpu.SemaphoreType.DMA,
        ],
    )
    out = emb(x.astype(jnp.int32), table)
    return out.reshape(batch, seq, d_model)


# speedup vs baseline: 1.0507x; 1.0247x over previous
import functools

import jax
import jax.numpy as jnp
from jax import lax
from jax.experimental import pallas as pl
from jax.experimental.pallas import tpu as pltpu
from jax.experimental.pallas import tpu_sc as plsc

_NUM_WORKERS = 32
_CHUNKS = (64, 56, 64, 56, 16)


def _emb_kernel_body(rows_per_worker, seq, x_hbm, table_hbm, out_hbm,
                     idx_v, rows0, rows1, sem0, sem1):
    wid = lax.axis_index("s") * 2 + lax.axis_index("c")
    base = wid * rows_per_worker
    w_per_row = seq // rows_per_worker
    pltpu.sync_copy(
        x_hbm.at[wid // w_per_row,
                 pl.ds((wid % w_per_row) * rows_per_worker, rows_per_worker)],
        idx_v)

    bufs = (rows0, rows1)
    sems = (sem0, sem1)
    offs = [0]
    for ch in _CHUNKS[:-1]:
        offs.append(offs[-1] + ch)

    def gather(c):
        b = c % 2
        return pltpu.async_copy(
            table_hbm.at[idx_v.at[pl.ds(offs[c], _CHUNKS[c])]],
            bufs[b].at[pl.ds(0, _CHUNKS[c])], sems[b])

    handles = [None, None]
    handles[0] = gather(0)
    for c in range(len(_CHUNKS)):
        if c + 1 < len(_CHUNKS):
            handles[(c + 1) % 2] = gather(c + 1)
        handles[c % 2].wait()
        pltpu.sync_copy(bufs[c % 2].at[pl.ds(0, _CHUNKS[c])],
                        out_hbm.at[pl.ds(base + offs[c], _CHUNKS[c])])


def kernel(x, table):
    batch, seq = x.shape
    _, d_model = table.shape
    n = batch * seq
    rows_per_worker = n // _NUM_WORKERS
    assert sum(_CHUNKS) == rows_per_worker

    mesh = plsc.VectorSubcoreMesh(core_axis_name="c", subcore_axis_name="s")
    emb = pl.kernel(
        functools.partial(_emb_kernel_body, rows_per_worker, seq),
        mesh=mesh,
        out_type=jax.ShapeDtypeStruct((n, d_model), jnp.float32),
        scratch_types=[
            pltpu.VMEM((rows_per_worker,), jnp.int32),
            pltpu.VMEM((max(_CHUNKS), d_model), jnp.float32),
            pltpu.VMEM((max(_CHUNKS[1::2]), d_model), jnp.float32),
            pltpu.SemaphoreType.DMA,
            pltpu.SemaphoreType.DMA,
        ],
    )
    out = emb(x.astype(jnp.int32), table)
    return out.reshape(batch, seq, d_model)
